# baseline (speedup 1.0000x reference)
import jax
import jax.numpy as jnp
from jax.experimental import pallas as pl
from jax.experimental.pallas import tpu as pltpu

_NCHUNK = 8


def _body(start_ref, kc_ref, vc_ref, key_ref, val_ref, ko_ref, vo_ref,
          bulk_sem, scat_sem):
    BG, T, H = kc_ref.shape
    NUM = key_ref.shape[1]
    start = start_ref[0]
    cb = BG // _NCHUNK

    bulk = []
    for buf, (src, dst) in enumerate(((kc_ref, ko_ref), (vc_ref, vo_ref))):
        for c in range(_NCHUNK):
            cp = pltpu.make_async_copy(
                src.at[pl.ds(c * cb, cb)],
                dst.at[pl.ds(c * cb, cb)],
                bulk_sem.at[buf, c],
            )
            cp.start()
            bulk.append(cp)
    for cp in bulk:
        cp.wait()

    no_wrap = start <= T - NUM

    @pl.when(no_wrap)
    def _():
        scat = []
        for buf, (src, dst) in enumerate(((key_ref, ko_ref), (val_ref, vo_ref))):
            cp = pltpu.make_async_copy(
                src,
                dst.at[:, pl.ds(start, NUM), :],
                scat_sem.at[buf, 0],
            )
            cp.start()
            scat.append(cp)
        for cp in scat:
            cp.wait()

    @pl.when(jnp.logical_not(no_wrap))
    def _():
        scat = []
        for buf, (src, dst) in enumerate(((key_ref, ko_ref), (val_ref, vo_ref))):
            for i in range(NUM):
                slot = start + i
                slot = jnp.where(slot >= T, slot - T, slot)
                cp = pltpu.make_async_copy(
                    src.at[:, pl.ds(i, 1), :],
                    dst.at[:, pl.ds(slot, 1), :],
                    scat_sem.at[buf, i],
                )
                cp.start()
                scat.append(cp)
        for cp in scat:
            cp.wait()


def kernel(key, value, k_cache, v_cache, input_pos):
    B, G, NUM, H = key.shape
    T = k_cache.shape[2]
    BG = B * G

    key_r = key.reshape(BG, NUM, H)
    val_r = value.reshape(BG, NUM, H)
    kc_r = k_cache.reshape(BG, T, H)
    vc_r = v_cache.reshape(BG, T, H)
    start = (jnp.asarray(input_pos, jnp.int32) % T).reshape(1)

    grid_spec = pltpu.PrefetchScalarGridSpec(
        num_scalar_prefetch=1,
        grid=(1,),
        in_specs=[pl.BlockSpec(memory_space=pl.ANY)] * 4,
        out_specs=[pl.BlockSpec(memory_space=pl.ANY)] * 2,
        scratch_shapes=[
            pltpu.SemaphoreType.DMA((2, _NCHUNK)),
            pltpu.SemaphoreType.DMA((2, NUM)),
        ],
    )
    ko, vo = pl.pallas_call(
        _body,
        grid_spec=grid_spec,
        out_shape=[jax.ShapeDtypeStruct((BG, T, H), jnp.float32)] * 2,
    )(start, kc_r, vc_r, key_r, val_r)
    return ko.reshape(B, G, T, H), vo.reshape(B, G, T, H)


# speedup vs baseline: 47.1998x; 47.1998x over previous
import jax
import jax.numpy as jnp
from jax.experimental import pallas as pl
from jax.experimental.pallas import tpu as pltpu

_BBG = 2


def _body(start_ref, kc_ref, vc_ref, key_ref, val_ref, ko_ref, vo_ref):
    T = kc_ref.shape[1]
    NUM = key_ref.shape[1]
    start = start_ref[0]

    row = jax.lax.broadcasted_iota(jnp.int32, (T, NUM), 0)
    col = jax.lax.broadcasted_iota(jnp.int32, (T, NUM), 1)
    j = row - start
    j = jnp.where(j < 0, j + T, j)
    onehot = (j == col).astype(jnp.float32)
    mask = (j < NUM)[:, :1]

    for b in range(_BBG):
        sub_k = jnp.dot(onehot, key_ref[b], preferred_element_type=jnp.float32)
        ko_ref[b] = jnp.where(mask, sub_k, kc_ref[b])
        sub_v = jnp.dot(onehot, val_ref[b], preferred_element_type=jnp.float32)
        vo_ref[b] = jnp.where(mask, sub_v, vc_ref[b])


def kernel(key, value, k_cache, v_cache, input_pos):
    B, G, NUM, H = key.shape
    T = k_cache.shape[2]
    BG = B * G

    key_r = key.reshape(BG, NUM, H)
    val_r = value.reshape(BG, NUM, H)
    kc_r = k_cache.reshape(BG, T, H)
    vc_r = v_cache.reshape(BG, T, H)
    start = (jnp.asarray(input_pos, jnp.int32) % T).reshape(1)

    grid_spec = pltpu.PrefetchScalarGridSpec(
        num_scalar_prefetch=1,
        grid=(BG // _BBG,),
        in_specs=[
            pl.BlockSpec((_BBG, T, H), lambda i, s: (i, 0, 0)),
            pl.BlockSpec((_BBG, T, H), lambda i, s: (i, 0, 0)),
            pl.BlockSpec((_BBG, NUM, H), lambda i, s: (i, 0, 0)),
            pl.BlockSpec((_BBG, NUM, H), lambda i, s: (i, 0, 0)),
        ],
        out_specs=[
            pl.BlockSpec((_BBG, T, H), lambda i, s: (i, 0, 0)),
            pl.BlockSpec((_BBG, T, H), lambda i, s: (i, 0, 0)),
        ],
    )
    ko, vo = pl.pallas_call(
        _body,
        grid_spec=grid_spec,
        out_shape=[jax.ShapeDtypeStruct((BG, T, H), jnp.float32)] * 2,
    )(start, kc_r, vc_r, key_r, val_r)
    return ko.reshape(B, G, T, H), vo.reshape(B, G, T, H)
